# baseline (speedup 1.0000x reference)
import functools

import jax
import jax.numpy as jnp
from jax.experimental import pallas as pl
from jax.experimental.pallas import tpu as pltpu

_THRESH = 0.1
_BM = 2048
_BN = 512
_L = 128
_UNROLL = 4


def _gather_kernel(nrows, xtab_ref, ytab_ref, i0_ref, i1_ref,
                   xn_ref, yn_ref, dh_ref):
    xt = xtab_ref[...]
    yt = ytab_ref[...]

    def gather2(idx):
        r_id = idx >> 7
        c = idx & 127
        gx = jnp.zeros(idx.shape, jnp.float32)
        gy = jnp.zeros(idx.shape, jnp.float32)
        for r in range(nrows):
            rowx = jnp.broadcast_to(xt[r:r + 1, :], idx.shape)
            rowy = jnp.broadcast_to(yt[r:r + 1, :], idx.shape)
            px = jnp.take_along_axis(rowx, c, axis=1)
            py = jnp.take_along_axis(rowy, c, axis=1)
            m = r_id == r
            gx = jnp.where(m, px, gx)
            gy = jnp.where(m, py, gy)
        return gx, gy

    x0, y0 = gather2(i0_ref[0])
    x1, y1 = gather2(i1_ref[0])
    dx = x1 - x0
    dy = y1 - y0
    n2 = dx * dx + dy * dy
    inv = 1.0 / jnp.maximum(jnp.sqrt(n2), 1e-6)
    xn_ref[...] = (dx * inv).reshape(1, _BM // _L, _L)
    yn_ref[...] = (dy * inv).reshape(1, _BM // _L, _L)
    q = n2 * inv * inv
    hf = jnp.where(q > _THRESH, 1.0, 0.0)
    dh_ref[...] = (hf[:8] + hf[8:]).reshape(1, 8, _L)


def _chunk(a_ref, bn_ref, idx):
    b = bn_ref[:, pl.ds(idx, _BN)]
    t32 = jax.lax.dot_general(a_ref[...], b, (((1,), (0,)), ((), ())),
                              preferred_element_type=jnp.float32)
    t = t32.astype(jnp.bfloat16)
    hf = jnp.where(jnp.abs(t) > jnp.bfloat16(_THRESH),
                   jnp.bfloat16(1.0), jnp.bfloat16(0.0))
    m = _BM
    while m > 16:
        m //= 2
        hf = hf[:m] + hf[m:]
    return hf.astype(jnp.float32)


def _count_kernel(nchunks, an_ref, bn_ref, out_ref, acc_ref):
    bi = pl.program_id(0)
    acc_ref[...] = jnp.zeros_like(acc_ref)

    def body(c, carry):
        base = pl.multiple_of(c * _UNROLL * _BN, _UNROLL * _BN)
        total = _chunk(an_ref, bn_ref, base)
        for u in range(1, _UNROLL):
            total = total + _chunk(an_ref, bn_ref, base + u * _BN)
        w = jnp.where(c == bi, 1.0, 2.0)
        acc_ref[...] += w * total
        return carry

    jax.lax.fori_loop(bi, nchunks // _UNROLL, body, 0)
    out_ref[...] = acc_ref[...].reshape(1, 16, _BN)


@jax.jit
def kernel(node_pos, edge_index):
    e = edge_index.shape[1]
    n = node_pos.shape[0]
    g = e // _BM
    nrows = n // _L
    xtab = node_pos[:, 0].reshape(nrows, _L)
    ytab = node_pos[:, 1].reshape(nrows, _L)
    idx = edge_index.astype(jnp.int32)
    i0 = idx[0].reshape(g, _BM // _L, _L)
    i1 = idx[1].reshape(g, _BM // _L, _L)

    xn3, yn3, dh = pl.pallas_call(
        functools.partial(_gather_kernel, nrows),
        grid=(g,),
        in_specs=[
            pl.BlockSpec((nrows, _L), lambda i: (0, 0)),
            pl.BlockSpec((nrows, _L), lambda i: (0, 0)),
            pl.BlockSpec((1, _BM // _L, _L), lambda i: (i, 0, 0)),
            pl.BlockSpec((1, _BM // _L, _L), lambda i: (i, 0, 0)),
        ],
        out_specs=[
            pl.BlockSpec((1, _BM // _L, _L), lambda i: (i, 0, 0)),
            pl.BlockSpec((1, _BM // _L, _L), lambda i: (i, 0, 0)),
            pl.BlockSpec((1, 8, _L), lambda i: (i, 0, 0)),
        ],
        out_shape=[
            jax.ShapeDtypeStruct((g, _BM // _L, _L), jnp.float32),
            jax.ShapeDtypeStruct((g, _BM // _L, _L), jnp.float32),
            jax.ShapeDtypeStruct((g, 8, _L), jnp.float32),
        ],
        compiler_params=pltpu.CompilerParams(
            dimension_semantics=("arbitrary",)),
    )(xtab, ytab, i0, i1)

    xnf = xn3.reshape(e)
    ynf = yn3.reshape(e)
    an = jnp.concatenate(
        [xnf[:, None], ynf[:, None], jnp.zeros((e, _L - 2), jnp.float32)],
        axis=1).astype(jnp.bfloat16)
    bn = jnp.concatenate(
        [xnf[None, :], ynf[None, :], jnp.zeros((_L - 2, e), jnp.float32)],
        axis=0).astype(jnp.bfloat16)

    out = pl.pallas_call(
        functools.partial(_count_kernel, e // _BN),
        grid=(g,),
        in_specs=[
            pl.BlockSpec((_BM, _L), lambda i: (i, 0)),
            pl.BlockSpec((_L, e), lambda i: (0, 0)),
        ],
        out_specs=pl.BlockSpec((1, 16, _BN), lambda i: (i, 0, 0)),
        out_shape=jax.ShapeDtypeStruct((g, 16, _BN), jnp.float32),
        scratch_shapes=[pltpu.VMEM((16, _BN), jnp.float32)],
        compiler_params=pltpu.CompilerParams(
            dimension_semantics=("arbitrary",)),
    )(an, bn)

    total = jnp.sum(out)
    diag = jnp.sum(dh)
    denom = e * (e - 1) / 2
    return (total - diag) * 0.5 / denom


# speedup vs baseline: 1.0011x; 1.0011x over previous
import functools

import jax
import jax.numpy as jnp
from jax.experimental import pallas as pl
from jax.experimental.pallas import tpu as pltpu

_THRESH = 0.1
_BM = 2048
_BN = 256
_L = 128
_UNROLL = 8


def _gather_kernel(nrows, xtab_ref, ytab_ref, i0_ref, i1_ref,
                   xn_ref, yn_ref, dh_ref):
    xt = xtab_ref[...]
    yt = ytab_ref[...]

    def gather2(idx):
        r_id = idx >> 7
        c = idx & 127
        gx = jnp.zeros(idx.shape, jnp.float32)
        gy = jnp.zeros(idx.shape, jnp.float32)
        for r in range(nrows):
            rowx = jnp.broadcast_to(xt[r:r + 1, :], idx.shape)
            rowy = jnp.broadcast_to(yt[r:r + 1, :], idx.shape)
            px = jnp.take_along_axis(rowx, c, axis=1)
            py = jnp.take_along_axis(rowy, c, axis=1)
            m = r_id == r
            gx = jnp.where(m, px, gx)
            gy = jnp.where(m, py, gy)
        return gx, gy

    x0, y0 = gather2(i0_ref[0])
    x1, y1 = gather2(i1_ref[0])
    dx = x1 - x0
    dy = y1 - y0
    n2 = dx * dx + dy * dy
    inv = 1.0 / jnp.maximum(jnp.sqrt(n2), 1e-6)
    xn_ref[...] = (dx * inv).reshape(1, _BM // _L, _L)
    yn_ref[...] = (dy * inv).reshape(1, _BM // _L, _L)
    q = n2 * inv * inv
    hf = jnp.where(q > _THRESH, 1.0, 0.0)
    dh_ref[...] = (hf[:8] + hf[8:]).reshape(1, 8, _L)


def _chunk(a_ref, bn_ref, idx):
    b = bn_ref[:, pl.ds(idx, _BN)]
    t32 = jax.lax.dot_general(a_ref[...], b, (((1,), (0,)), ((), ())),
                              preferred_element_type=jnp.float32)
    t = t32.astype(jnp.bfloat16)
    hf = jnp.where(jnp.abs(t) > jnp.bfloat16(_THRESH),
                   jnp.bfloat16(1.0), jnp.bfloat16(0.0))
    m = _BM
    while m > 16:
        m //= 2
        hf = hf[:m] + hf[m:]
    return hf.astype(jnp.float32)


def _count_kernel(nchunks, an_ref, bn_ref, out_ref, acc_ref):
    bi = pl.program_id(0)
    acc_ref[...] = jnp.zeros_like(acc_ref)

    def body(c, carry):
        base = pl.multiple_of(c * _UNROLL * _BN, _UNROLL * _BN)
        total = _chunk(an_ref, bn_ref, base)
        for u in range(1, _UNROLL):
            total = total + _chunk(an_ref, bn_ref, base + u * _BN)
        w = jnp.where(c == bi, 1.0, 2.0)
        acc_ref[...] += w * total
        return carry

    jax.lax.fori_loop(bi, nchunks // _UNROLL, body, 0)
    out_ref[...] = acc_ref[...].reshape(1, 16, _BN)


@jax.jit
def kernel(node_pos, edge_index):
    e = edge_index.shape[1]
    n = node_pos.shape[0]
    g = e // _BM
    nrows = n // _L
    xtab = node_pos[:, 0].reshape(nrows, _L)
    ytab = node_pos[:, 1].reshape(nrows, _L)
    idx = edge_index.astype(jnp.int32)
    i0 = idx[0].reshape(g, _BM // _L, _L)
    i1 = idx[1].reshape(g, _BM // _L, _L)

    xn3, yn3, dh = pl.pallas_call(
        functools.partial(_gather_kernel, nrows),
        grid=(g,),
        in_specs=[
            pl.BlockSpec((nrows, _L), lambda i: (0, 0)),
            pl.BlockSpec((nrows, _L), lambda i: (0, 0)),
            pl.BlockSpec((1, _BM // _L, _L), lambda i: (i, 0, 0)),
            pl.BlockSpec((1, _BM // _L, _L), lambda i: (i, 0, 0)),
        ],
        out_specs=[
            pl.BlockSpec((1, _BM // _L, _L), lambda i: (i, 0, 0)),
            pl.BlockSpec((1, _BM // _L, _L), lambda i: (i, 0, 0)),
            pl.BlockSpec((1, 8, _L), lambda i: (i, 0, 0)),
        ],
        out_shape=[
            jax.ShapeDtypeStruct((g, _BM // _L, _L), jnp.float32),
            jax.ShapeDtypeStruct((g, _BM // _L, _L), jnp.float32),
            jax.ShapeDtypeStruct((g, 8, _L), jnp.float32),
        ],
        compiler_params=pltpu.CompilerParams(
            dimension_semantics=("arbitrary",)),
    )(xtab, ytab, i0, i1)

    xnf = xn3.reshape(e)
    ynf = yn3.reshape(e)
    an = jnp.concatenate(
        [xnf[:, None], ynf[:, None], jnp.zeros((e, _L - 2), jnp.float32)],
        axis=1).astype(jnp.bfloat16)
    bn = jnp.concatenate(
        [xnf[None, :], ynf[None, :], jnp.zeros((_L - 2, e), jnp.float32)],
        axis=0).astype(jnp.bfloat16)

    out = pl.pallas_call(
        functools.partial(_count_kernel, e // _BN),
        grid=(g,),
        in_specs=[
            pl.BlockSpec((_BM, _L), lambda i: (i, 0)),
            pl.BlockSpec((_L, e), lambda i: (0, 0)),
        ],
        out_specs=pl.BlockSpec((1, 16, _BN), lambda i: (i, 0, 0)),
        out_shape=jax.ShapeDtypeStruct((g, 16, _BN), jnp.float32),
        scratch_shapes=[pltpu.VMEM((16, _BN), jnp.float32)],
        compiler_params=pltpu.CompilerParams(
            dimension_semantics=("arbitrary",)),
    )(an, bn)

    total = jnp.sum(out)
    diag = jnp.sum(dh)
    denom = e * (e - 1) / 2
    return (total - diag) * 0.5 / denom
